# baseline (device time: 112484 ns/iter reference)
import jax
import jax.numpy as jnp
from jax import lax
from jax.experimental import pallas as pl
from jax.experimental.pallas import tpu as pltpu


def kernel(x, dy):
    m, d = x.shape
    _, f = dy.shape
    d_half = d // 2
    f_half = f // 2

    def body(x_ref, dy_ref, out_ref, xsend_ref, xrecv_ref, send_sems, recv_sems):
        my_x = lax.axis_index("x")
        my_y = lax.axis_index("y")
        peer_x = 1 - my_x
        peer_y = 1 - my_y

        barrier = pltpu.get_barrier_semaphore()
        pl.semaphore_signal(barrier, inc=1, device_id=(peer_x, my_y),
                            device_id_type=pl.DeviceIdType.MESH)
        pl.semaphore_signal(barrier, inc=1, device_id=(my_x, peer_y),
                            device_id_type=pl.DeviceIdType.MESH)
        pl.semaphore_wait(barrier, 2)

        dims = (((0,), (0,)), ((), ()))
        dy_half = dy_ref[:, pl.ds(my_y * f_half, f_half)]

        xsend_ref[...] = lax.dot_general(
            x_ref[:, pl.ds(peer_x * d_half, d_half)], dy_half, dims,
            preferred_element_type=jnp.float32)

        rdma_x = pltpu.make_async_remote_copy(
            src_ref=xsend_ref, dst_ref=xrecv_ref,
            send_sem=send_sems.at[0], recv_sem=recv_sems.at[0],
            device_id=(peer_x, my_y), device_id_type=pl.DeviceIdType.MESH)
        rdma_x.start()

        mine = lax.dot_general(
            x_ref[:, pl.ds(my_x * d_half, d_half)], dy_half, dims,
            preferred_element_type=jnp.float32)

        rdma_x.wait()
        out_ref[:, pl.ds(my_y * f_half, f_half)] = mine + xrecv_ref[...]

        rdma_y = pltpu.make_async_remote_copy(
            src_ref=out_ref.at[:, pl.ds(my_y * f_half, f_half)],
            dst_ref=out_ref.at[:, pl.ds(my_y * f_half, f_half)],
            send_sem=send_sems.at[1], recv_sem=recv_sems.at[1],
            device_id=(my_x, peer_y), device_id_type=pl.DeviceIdType.MESH)
        rdma_y.start()
        rdma_y.wait()

    return pl.pallas_call(
        body,
        out_shape=jax.ShapeDtypeStruct((d_half, f), jnp.float32),
        in_specs=[pl.BlockSpec(memory_space=pltpu.VMEM),
                  pl.BlockSpec(memory_space=pltpu.VMEM)],
        out_specs=pl.BlockSpec(memory_space=pltpu.VMEM),
        scratch_shapes=[
            pltpu.VMEM((d_half, f_half), jnp.float32),
            pltpu.VMEM((d_half, f_half), jnp.float32),
            pltpu.SemaphoreType.DMA((2,)),
            pltpu.SemaphoreType.DMA((2,)),
        ],
        compiler_params=pltpu.CompilerParams(collective_id=0),
    )(x, dy)


# device time: 68379 ns/iter; 1.6450x vs baseline; 1.6450x over previous
import jax
import jax.numpy as jnp
from jax import lax
from jax.experimental import pallas as pl
from jax.experimental.pallas import tpu as pltpu

K = 16


def kernel(x, dy):
    m, d = x.shape
    _, f = dy.shape
    d_half = d // 2
    f_half = f // 2
    cw = f_half // K

    def body(x_ref, dy_ref, out_ref, xsend_ref, xrecv_ref,
             xsend_sems, xrecv_sems, ysend_sems, yrecv_sems):
        my_x = lax.axis_index("x")
        my_y = lax.axis_index("y")
        peer_x = 1 - my_x
        peer_y = 1 - my_y

        barrier = pltpu.get_barrier_semaphore()
        pl.semaphore_signal(barrier, inc=1, device_id=(peer_x, my_y),
                            device_id_type=pl.DeviceIdType.MESH)
        pl.semaphore_signal(barrier, inc=1, device_id=(my_x, peer_y),
                            device_id_type=pl.DeviceIdType.MESH)
        pl.semaphore_wait(barrier, 2)

        dims = (((0,), (0,)), ((), ()))
        dy_half = dy_ref[:, pl.ds(my_y * f_half, f_half)]

        xsend_ref[...] = lax.dot_general(
            x_ref[:, pl.ds(peer_x * d_half, d_half)], dy_half, dims,
            preferred_element_type=jnp.float32)

        rdma_x = []
        for k in range(K):
            r = pltpu.make_async_remote_copy(
                src_ref=xsend_ref.at[:, k * cw:(k + 1) * cw],
                dst_ref=xrecv_ref.at[:, k * cw:(k + 1) * cw],
                send_sem=xsend_sems.at[k], recv_sem=xrecv_sems.at[k],
                device_id=(peer_x, my_y),
                device_id_type=pl.DeviceIdType.MESH)
            r.start()
            rdma_x.append(r)

        out_ref[:, pl.ds(my_y * f_half, f_half)] = lax.dot_general(
            x_ref[:, pl.ds(my_x * d_half, d_half)], dy_half, dims,
            preferred_element_type=jnp.float32)

        rdma_y = []
        for k in range(K):
            rdma_x[k].wait_recv()
            sl = pl.ds(my_y * f_half + k * cw, cw)
            out_ref[:, sl] = out_ref[:, sl] + xrecv_ref[:, k * cw:(k + 1) * cw]
            r = pltpu.make_async_remote_copy(
                src_ref=out_ref.at[:, sl],
                dst_ref=out_ref.at[:, sl],
                send_sem=ysend_sems.at[k], recv_sem=yrecv_sems.at[k],
                device_id=(my_x, peer_y),
                device_id_type=pl.DeviceIdType.MESH)
            r.start()
            rdma_y.append(r)

        for k in range(K):
            rdma_y[k].wait_recv()
        for k in range(K):
            rdma_x[k].wait_send()
            rdma_y[k].wait_send()

    return pl.pallas_call(
        body,
        out_shape=jax.ShapeDtypeStruct((d_half, f), jnp.float32),
        in_specs=[pl.BlockSpec(memory_space=pltpu.VMEM),
                  pl.BlockSpec(memory_space=pltpu.VMEM)],
        out_specs=pl.BlockSpec(memory_space=pltpu.VMEM),
        scratch_shapes=[
            pltpu.VMEM((d_half, f_half), jnp.float32),
            pltpu.VMEM((d_half, f_half), jnp.float32),
            pltpu.SemaphoreType.DMA((K,)),
            pltpu.SemaphoreType.DMA((K,)),
            pltpu.SemaphoreType.DMA((K,)),
            pltpu.SemaphoreType.DMA((K,)),
        ],
        compiler_params=pltpu.CompilerParams(collective_id=0),
    )(x, dy)
